# baseline (speedup 1.0000x reference)
import jax
import jax.numpy as jnp
from jax import lax
from jax.experimental import pallas as pl
from jax.experimental.pallas import tpu as pltpu
from jax.experimental.pallas import tpu_sc as plsc

_B = 4096
_T = 200
_D = 64
_V = 100000
_NC, _NS = 2, 16
_NW = _NC * _NS
_BW = _B // _NW
_NBG = 4
_NBS = 2
_LA = 2


def _body(tob0, tob1, tob2, tob3, iob0, iob1, iob2, iob3,
          tst0, tst1, tst2, ist0, ist1, ist2, vr0, vr1, wb,
          s_out, o_out,
          idx_v, rows_v, tr_v, wb_v,
          g0, g1, g2, g3, s0, s1):
    gsem = (g0, g1, g2, g3)
    osem = (s0, s1)
    wid = lax.axis_index("s") * _NC + lax.axis_index("c")
    b0 = wid * _BW

    lanes = jnp.arange(16, dtype=jnp.int32)
    stages = (1, 2, 4, 8)
    tmask = [(lanes & s) == 0 for s in stages]
    tidxm = [(lanes - s) % 16 for s in stages]
    tidxp = [(lanes + s) % 16 for s in stages]

    dnums = lax.GatherDimensionNumbers(
        offset_dims=(), collapsed_slice_dims=(0,), start_index_map=(0,))

    def lperm(x, idx):
        return lax.gather(x, idx[:, None], dnums, (1,),
                          mode=lax.GatherScatterMode.PROMISE_IN_BOUNDS)

    def transpose_block(bg, bs):
        @pl.loop(0, 4)
        def _dc(dc):
            @pl.loop(0, 8)
            def _jj(j):
                regs = [rows_v[bg, 16 * j + k, pl.ds(16 * dc, 16)]
                        for k in range(16)]
                for si, s in enumerate(stages):
                    m, im, ip = tmask[si], tidxm[si], tidxp[si]
                    for i in range(16):
                        if i & s:
                            continue
                        jo = i | s
                        a, b = regs[i], regs[jo]
                        regs[i] = jnp.where(m, a, lperm(b, im))
                        regs[jo] = jnp.where(m, lperm(a, ip), b)
                for k in range(16):
                    d = 16 * dc + k
                    tr_v[bs, 0, 0, d // 8, 0, d % 8, pl.ds(16 * j, 16)] = regs[k]

    for i, (tbl, sidx) in enumerate(((tst0, ist0), (tst1, ist1), (tst2, ist2))):
        pltpu.sync_copy(sidx, idx_v.at[pl.ds(0, _NW)])
        pltpu.async_copy(tbl.at[idx_v.at[wid]], rows_v.at[0],
                         gsem[0]).wait()
        transpose_block(0, 0)
        pltpu.sync_copy(tr_v.at[0, 0],
                        s_out.at[pl.ds(i, 1), :, pl.ds(wid, 1), :, :])

    for f, (tbl, iob) in enumerate(((tob0, iob0), (tob1, iob1),
                                    (tob2, iob2), (tob3, iob3))):
        pltpu.sync_copy(iob.at[:, pl.ds(b0, _BW)], idx_v)

        def fire(t, bg, tbl=tbl):
            pltpu.async_copy(tbl.at[idx_v.at[t]], rows_v.at[bg], gsem[bg])

        def wait_g(bg, tbl=tbl):
            pltpu.make_async_copy(tbl.at[pl.ds(0, _BW)], rows_v.at[bg],
                                  gsem[bg]).wait()

        def fire_store(t, bs, f=f):
            pltpu.async_copy(
                tr_v.at[bs],
                o_out.at[pl.ds(t, 1), pl.ds(f, 1), :, pl.ds(wid, 1), :, :],
                osem[bs])

        def wait_s(bs, f=f):
            pltpu.make_async_copy(
                tr_v.at[bs],
                o_out.at[pl.ds(0, 1), pl.ds(f, 1), :, pl.ds(wid, 1), :, :],
                osem[bs]).wait()

        fire(0, 0)
        fire(1, 1)

        @pl.loop(0, _T, step=_NBG)
        def _blocks(tt):
            for b in range(_NBG):
                t = tt + b
                bs = b % _NBS
                wait_g(b)

                @pl.when(t + _LA < _T)
                def _():
                    fire(t + _LA, (b + _LA) % _NBG)

                @pl.when(t >= _NBS)
                def _():
                    wait_s(bs)

                transpose_block(b, bs)
                fire_store(t, bs)

        for b in range(_NBS):
            wait_s(b)

    pltpu.sync_copy(wb, wb_v)
    for i, vob in enumerate((vr0, vr1)):
        pltpu.sync_copy(vob.at[:, pl.ds(b0, _BW)], idx_v)

        def fire_store_r(t, bs, i=i):
            pltpu.async_copy(
                tr_v.at[bs],
                o_out.at[pl.ds(t, 1), pl.ds(4 + i, 1), :, pl.ds(wid, 1), :, :],
                osem[bs])

        def wait_s_r(bs, i=i):
            pltpu.make_async_copy(
                tr_v.at[bs],
                o_out.at[pl.ds(0, 1), pl.ds(4 + i, 1), :, pl.ds(wid, 1), :, :],
                osem[bs]).wait()

        @pl.loop(0, _T, step=_NBS)
        def _blocks(tt):
            for b in range(_NBS):
                t = tt + b

                @pl.when(t >= _NBS)
                def _():
                    wait_s_r(b)

                vj = [plsc.bitcast(idx_v[t, pl.ds(16 * j, 16)], jnp.float32)
                      for j in range(8)]

                @pl.loop(0, 4)
                def _dgrp(dc, i=i, b=b, vj=vj):
                    wv = wb_v[pl.ds(i * 2 * _D + dc * 16, 16)]
                    bv = wb_v[pl.ds(i * 2 * _D + _D + dc * 16, 16)]
                    for l in range(16):
                        ws = wv[l]
                        bs_ = bv[l]
                        for j in range(8):
                            d = dc * 16 + l
                            tr_v[b, 0, 0, d // 8, 0, d % 8,
                                 pl.ds(16 * j, 16)] = vj[j] * ws + bs_

                fire_store_r(t, b)

        for b in range(_NBS):
            wait_s_r(b)


_mesh = plsc.VectorSubcoreMesh(core_axis_name="c", subcore_axis_name="s",
                               num_cores=_NC, num_subcores=_NS)

_out_type = (
    jax.ShapeDtypeStruct((3, _D // 8, _B // 128, 8, 128), jnp.float32),
    jax.ShapeDtypeStruct((_T, 6, _D // 8, _B // 128, 8, 128), jnp.float32),
)

_scratch = [
    pltpu.VMEM((_T, _BW), jnp.int32),
    pltpu.VMEM((_NBG, _BW, _D), jnp.float32),
    pltpu.VMEM((_NBS, 1, 1, _D // 8, 1, 8, _BW), jnp.float32),
    pltpu.VMEM((4 * _D,), jnp.float32),
] + [pltpu.SemaphoreType.DMA] * 6

_sc_call = pl.kernel(_body, out_type=_out_type, mesh=_mesh,
                     scratch_types=_scratch,
                     compiler_params=pltpu.CompilerParams(
                         use_tc_tiling_on_sc=False,
                         needs_layout_passes=False))


def kernel(static_cat_0, table_static_0, static_cat_1, table_static_1,
           static_cat_2, table_static_2,
           obs_cat_0, table_obs_0, obs_cat_1, table_obs_1,
           obs_cat_2, table_obs_2, obs_cat_3, table_obs_3,
           obs_real_0, w_real_0, b_real_0, obs_real_1, w_real_1, b_real_1):
    iob = [x.T for x in (obs_cat_0, obs_cat_1, obs_cat_2, obs_cat_3)]
    ist = [x.reshape(_NW, _BW) for x in
           (static_cat_0, static_cat_1, static_cat_2)]
    vr = [lax.bitcast_convert_type(x.T, jnp.int32)
          for x in (obs_real_0, obs_real_1)]
    wb = jnp.concatenate([w_real_0, b_real_0, w_real_1, b_real_1])

    s_p, o_p = _sc_call(
        table_obs_0, table_obs_1, table_obs_2, table_obs_3,
        iob[0], iob[1], iob[2], iob[3],
        table_static_0, table_static_1, table_static_2,
        ist[0], ist[1], ist[2], vr[0], vr[1], wb)

    s = jnp.transpose(s_p, (0, 1, 3, 2, 4)).reshape(3, _D, _B)
    s = jnp.transpose(s, (2, 0, 1))
    o = jnp.transpose(o_p, (0, 1, 2, 4, 3, 5)).reshape(_T, 6, _D, _B)
    o = jnp.transpose(o, (3, 0, 1, 2))
    return (s, o)


# speedup vs baseline: 1.0181x; 1.0181x over previous
import jax
import jax.numpy as jnp
from jax import lax
from jax.experimental import pallas as pl
from jax.experimental.pallas import tpu as pltpu
from jax.experimental.pallas import tpu_sc as plsc

_B = 4096
_T = 200
_D = 64
_V = 100000
_NC, _NS = 2, 16
_NW = _NC * _NS
_BW = _B // _NW
_NBG = 4
_NBS = 4
_LA = 3


def _body(tob0, tob1, tob2, tob3, iob0, iob1, iob2, iob3,
          tst0, tst1, tst2, ist0, ist1, ist2, vr0, vr1, wb,
          s_out, o_out,
          idx_v, rows_v, tr_v, wb_v,
          g0, g1, g2, g3, s0, s1, s2, s3):
    gsem = (g0, g1, g2, g3)
    osem = (s0, s1, s2, s3)
    wid = lax.axis_index("s") * _NC + lax.axis_index("c")
    b0 = wid * _BW

    lanes = jnp.arange(16, dtype=jnp.int32)
    stages = (1, 2, 4, 8)
    tmask = [(lanes & s) == 0 for s in stages]
    tidxm = [(lanes - s) % 16 for s in stages]
    tidxp = [(lanes + s) % 16 for s in stages]

    dnums = lax.GatherDimensionNumbers(
        offset_dims=(), collapsed_slice_dims=(0,), start_index_map=(0,))

    def lperm(x, idx):
        return lax.gather(x, idx[:, None], dnums, (1,),
                          mode=lax.GatherScatterMode.PROMISE_IN_BOUNDS)

    def transpose_block(bg, bs):
        @pl.loop(0, 4)
        def _dc(dc):
            @pl.loop(0, 8)
            def _jj(j):
                regs = [rows_v[bg, 16 * j + k, pl.ds(16 * dc, 16)]
                        for k in range(16)]
                for si, s in enumerate(stages):
                    m, im, ip = tmask[si], tidxm[si], tidxp[si]
                    for i in range(16):
                        if i & s:
                            continue
                        jo = i | s
                        a, b = regs[i], regs[jo]
                        regs[i] = jnp.where(m, a, lperm(b, im))
                        regs[jo] = jnp.where(m, lperm(a, ip), b)
                for k in range(16):
                    tr_v[bs, 0, 0, 16 * dc + k, pl.ds(16 * j, 16)] = regs[k]

    for i, (tbl, sidx) in enumerate(((tst0, ist0), (tst1, ist1), (tst2, ist2))):
        pltpu.sync_copy(sidx, idx_v.at[pl.ds(0, _NW)])
        pltpu.async_copy(tbl.at[idx_v.at[wid]],
                         rows_v.at[0, :, pl.ds(0, 128)], gsem[0]).wait()
        transpose_block(0, 0)
        pltpu.sync_copy(tr_v.at[0, 0],
                        s_out.at[pl.ds(i, 1), :, pl.ds(b0, _BW)])

    for f, (tbl, iob) in enumerate(((tob0, iob0), (tob1, iob1),
                                    (tob2, iob2), (tob3, iob3))):
        pltpu.sync_copy(iob.at[:, pl.ds(b0, _BW)], idx_v)

        def fire(t, bg, tbl=tbl):
            pltpu.async_copy(tbl.at[idx_v.at[t]], rows_v.at[bg, :, pl.ds(0, 128)], gsem[bg])

        def wait_g(bg, tbl=tbl):
            pltpu.make_async_copy(tbl.at[pl.ds(0, _BW)],
                                  rows_v.at[bg, :, pl.ds(0, 128)],
                                  gsem[bg]).wait()

        def fire_store(t, bs, f=f):
            pltpu.async_copy(
                tr_v.at[bs],
                o_out.at[pl.ds(t, 1), pl.ds(f, 1), :, pl.ds(b0, _BW)],
                osem[bs])

        def wait_s(bs, f=f):
            pltpu.make_async_copy(
                tr_v.at[bs],
                o_out.at[pl.ds(0, 1), pl.ds(f, 1), :, pl.ds(b0, _BW)],
                osem[bs]).wait()

        fire(0, 0)
        fire(1, 1)
        fire(2, 2)

        @pl.loop(0, _T, step=_NBG)
        def _blocks(tt):
            for b in range(_NBG):
                t = tt + b
                bs = b % _NBS
                wait_g(b)

                @pl.when(t + _LA < _T)
                def _():
                    fire(t + _LA, (b + _LA) % _NBG)

                @pl.when(t >= _NBS)
                def _():
                    wait_s(bs)

                transpose_block(b, bs)
                fire_store(t, bs)

        for b in range(_NBS):
            wait_s(b)

    pltpu.sync_copy(wb, wb_v)
    for i, vob in enumerate((vr0, vr1)):
        pltpu.sync_copy(vob.at[:, pl.ds(b0, _BW)], idx_v)

        def fire_store_r(t, bs, i=i):
            pltpu.async_copy(
                tr_v.at[bs],
                o_out.at[pl.ds(t, 1), pl.ds(4 + i, 1), :, pl.ds(b0, _BW)],
                osem[bs])

        def wait_s_r(bs, i=i):
            pltpu.make_async_copy(
                tr_v.at[bs],
                o_out.at[pl.ds(0, 1), pl.ds(4 + i, 1), :, pl.ds(b0, _BW)],
                osem[bs]).wait()

        @pl.loop(0, _T, step=_NBS)
        def _blocks(tt):
            for b in range(_NBS):
                t = tt + b

                @pl.when(t >= _NBS)
                def _():
                    wait_s_r(b)

                vj = [plsc.bitcast(idx_v[t, pl.ds(16 * j, 16)], jnp.float32)
                      for j in range(8)]

                @pl.loop(0, 4)
                def _dgrp(dc, i=i, b=b, vj=vj):
                    wv = wb_v[pl.ds(i * 2 * _D + dc * 16, 16)]
                    bv = wb_v[pl.ds(i * 2 * _D + _D + dc * 16, 16)]
                    for l in range(16):
                        ws = wv[l]
                        bs_ = bv[l]
                        for j in range(8):
                            tr_v[b, 0, 0, dc * 16 + l, pl.ds(16 * j, 16)] = (
                                vj[j] * ws + bs_)

                fire_store_r(t, b)

        for b in range(_NBS):
            wait_s_r(b)


_mesh = plsc.VectorSubcoreMesh(core_axis_name="c", subcore_axis_name="s",
                               num_cores=_NC, num_subcores=_NS)

_out_type = (
    jax.ShapeDtypeStruct((3, _D, _B), jnp.float32),
    jax.ShapeDtypeStruct((_T, 6, _D, _B), jnp.float32),
)

_scratch = [
    pltpu.VMEM((_T, _BW), jnp.int32),
    pltpu.VMEM((_NBG, _BW, 128), jnp.float32),
    pltpu.VMEM((_NBS, 1, 1, _D, _BW), jnp.float32),
    pltpu.VMEM((4 * _D,), jnp.float32),
] + [pltpu.SemaphoreType.DMA] * 8

_sc_call = pl.kernel(_body, out_type=_out_type, mesh=_mesh,
                     scratch_types=_scratch,
                     compiler_params=pltpu.CompilerParams(
                         needs_layout_passes=False))


def kernel(static_cat_0, table_static_0, static_cat_1, table_static_1,
           static_cat_2, table_static_2,
           obs_cat_0, table_obs_0, obs_cat_1, table_obs_1,
           obs_cat_2, table_obs_2, obs_cat_3, table_obs_3,
           obs_real_0, w_real_0, b_real_0, obs_real_1, w_real_1, b_real_1):
    def padt(t):
        return jnp.pad(t, ((0, 0), (0, 128 - _D)))

    iob = [x.T for x in (obs_cat_0, obs_cat_1, obs_cat_2, obs_cat_3)]
    ist = [x.reshape(_NW, _BW) for x in
           (static_cat_0, static_cat_1, static_cat_2)]
    vr = [lax.bitcast_convert_type(x.T, jnp.int32)
          for x in (obs_real_0, obs_real_1)]
    wb = jnp.concatenate([w_real_0, b_real_0, w_real_1, b_real_1])

    s_p, o_p = _sc_call(
        padt(table_obs_0), padt(table_obs_1), padt(table_obs_2),
        padt(table_obs_3),
        iob[0], iob[1], iob[2], iob[3],
        padt(table_static_0), padt(table_static_1), padt(table_static_2),
        ist[0], ist[1], ist[2], vr[0], vr[1], wb)

    s = jnp.transpose(s_p, (2, 0, 1))
    o = jnp.transpose(o_p, (3, 0, 1, 2))
    return (s, o)
